# baseline (speedup 1.0000x reference)
import functools

import jax
import jax.numpy as jnp
from jax import lax
from jax.experimental import pallas as pl
from jax.experimental.pallas import tpu as pltpu
from jax.experimental.pallas import tpu_sc as plsc

B, S = 4096, 200
BS = B * S

VOCABS = [7, 7, 2, 3, 4, 2, 10, 3]
WIDTHS = [16, 16, 8, 8, 16, 8, 16, 16]
WOFFS = [0, 16, 32, 40, 48, 64, 72, 88]
NA, NB_ = 294, 240
STRIDES = [42 * NB_, 6 * NB_, 3 * NB_, NB_, 60, 30, 3, 1]
ADIGS = [(42, 7), (6, 7), (3, 2), (1, 3)]
BDIGS = [(60, 4), (30, 2), (3, 10), (1, 3)]

QA_ROWS_PER_STEP = 6


def _table_body(e0, e1, e2, e3, e4, e5, e6, e7, w_ref, b_ref, out_ref,
                qa_ref, qb_ref):
    es = [e0, e1, e2, e3, e4, e5, e6, e7]

    @pl.when(pl.program_id(0) == 0)
    def _build_quads():
        ps = []
        for t in range(8):
            ps.append(jnp.dot(es[t][...], w_ref[WOFFS[t]:WOFFS[t] + WIDTHS[t], :],
                              preferred_element_type=jnp.float32))
        qa = jnp.broadcast_to(b_ref[...], (NA, 128))
        for t, (stride, voc) in enumerate(ADIGS):
            r = lax.broadcasted_iota(jnp.int32, (NA, VOCABS[t]), 0)
            c = lax.broadcasted_iota(jnp.int32, (NA, VOCABS[t]), 1)
            sel = ((r // stride) % voc == c).astype(jnp.float32)
            qa = qa + jnp.dot(sel, ps[t], preferred_element_type=jnp.float32)
        qa_ref[...] = qa
        qb = jnp.zeros((NB_, 128), jnp.float32)
        for k, (stride, voc) in enumerate(BDIGS):
            t = 4 + k
            r = lax.broadcasted_iota(jnp.int32, (NB_, VOCABS[t]), 0)
            c = lax.broadcasted_iota(jnp.int32, (NB_, VOCABS[t]), 1)
            sel = ((r // stride) % voc == c).astype(jnp.float32)
            qb = qb + jnp.dot(sel, ps[t], preferred_element_type=jnp.float32)
        qb_ref[...] = qb

    i = pl.program_id(0)
    qa_rows = qa_ref[pl.ds(i * QA_ROWS_PER_STEP, QA_ROWS_PER_STEP), :]
    out_ref[...] = (qa_rows[:, None, :]
                    + qb_ref[...][None, :, :])


def _build_table(es, W, b2):
    in_specs = []
    for t in range(8):
        in_specs.append(pl.BlockSpec((VOCABS[t], WIDTHS[t]), lambda i: (0, 0)))
    in_specs.append(pl.BlockSpec((104, 128), lambda i: (0, 0)))
    in_specs.append(pl.BlockSpec((1, 128), lambda i: (0, 0)))
    return pl.pallas_call(
        _table_body,
        grid=(NA // QA_ROWS_PER_STEP,),
        in_specs=in_specs,
        out_specs=pl.BlockSpec((QA_ROWS_PER_STEP, NB_, 128), lambda i: (i, 0, 0)),
        out_shape=jax.ShapeDtypeStruct((NA, NB_, 128), jnp.float32),
        scratch_shapes=[pltpu.VMEM((NA, 128), jnp.float32),
                        pltpu.VMEM((NB_, 128), jnp.float32)],
        compiler_params=pltpu.CompilerParams(
            dimension_semantics=("arbitrary",)),
    )(*es, W, b2)


IDX_CN = 16384


def _idx_body(tokt_ref, out_ref):
    r = lax.broadcasted_iota(jnp.int32, (8, 1), 0)
    s = jnp.full((8, 1), STRIDES[7], jnp.int32)
    for t in range(7):
        s = jnp.where(r == t, STRIDES[t], s)
    prod = tokt_ref[...] * s
    idx = jnp.sum(prod, axis=0)
    out_ref[...] = idx.reshape(1, 1, IDX_CN)


def _build_idx(tokT):
    return pl.pallas_call(
        _idx_body,
        grid=(BS // IDX_CN,),
        in_specs=[pl.BlockSpec((8, IDX_CN), lambda i: (0, i))],
        out_specs=pl.BlockSpec((1, 1, IDX_CN), lambda i: (i, 0, 0)),
        out_shape=jax.ShapeDtypeStruct((BS // IDX_CN, 1, IDX_CN), jnp.int32),
    )(tokT)


NC, NS = 2, 16
NW = NC * NS
CH = 128
GRP = 4
TOK_PER_W = BS // NW
CROWS_PER_W = TOK_PER_W // CH
NGRP = CROWS_PER_W // GRP


def _sc_gather_body(bt_hbm, idx_hbm, out_hbm, idx_v, rows_v, gsem):
    wid = lax.axis_index("s") * NC + lax.axis_index("c")
    row0 = wid * CROWS_PER_W

    def grp_body(g, carry):
        base = row0 + g * GRP
        pltpu.sync_copy(idx_hbm.at[pl.ds(base, GRP), :], idx_v)
        handles = [pltpu.async_copy(bt_hbm.at[idx_v.at[j]],
                                    rows_v.at[pl.ds(j * CH, CH), :], gsem)
                   for j in range(GRP)]
        for h in handles:
            h.wait()
        pltpu.sync_copy(rows_v, out_hbm.at[pl.ds(base * CH, GRP * CH), :])
        return carry

    lax.fori_loop(0, NGRP, grp_body, 0)


def _sc_gather(bt, idx):
    mesh = plsc.VectorSubcoreMesh(core_axis_name="c", subcore_axis_name="s",
                                  num_cores=NC, num_subcores=NS)
    return pl.kernel(
        _sc_gather_body,
        out_type=jax.ShapeDtypeStruct((BS, 128), jnp.float32),
        mesh=mesh,
        scratch_types=[pltpu.VMEM((GRP, CH), jnp.int32),
                       pltpu.VMEM((GRP * CH, 128), jnp.float32),
                       pltpu.SemaphoreType.DMA],
    )(bt, idx)


def kernel(tokens, E0, E1, E2, E3, E4, E5, E6, E7, W, b):
    es = [E0, E1, E2, E3, E4, E5, E6, E7]
    bt = _build_table(es, W, b.reshape(1, 128)).reshape(NA * NB_, 128)
    tokT = tokens.reshape(BS, 8).T
    idx = _build_idx(tokT).reshape(BS // CH, CH)
    out = _sc_gather(bt, idx)
    return out.reshape(B, S, 128)


# speedup vs baseline: 1.1634x; 1.1634x over previous
import jax
import jax.numpy as jnp
from jax import lax
from jax.experimental import pallas as pl
from jax.experimental.pallas import tpu as pltpu
from jax.experimental.pallas import tpu_sc as plsc

B, S = 4096, 200
BS = B * S

VOCABS = [7, 7, 2, 3, 4, 2, 10, 3]
WIDTHS = [16, 16, 8, 8, 16, 8, 16, 16]
WOFFS = [0, 16, 32, 40, 48, 64, 72, 88]
NA, NB_ = 294, 240
STRIDES = [42 * NB_, 6 * NB_, 3 * NB_, NB_, 60, 30, 3, 1]
ADIGS = [(42, 7), (6, 7), (3, 2), (1, 3)]
BDIGS = [(60, 4), (30, 2), (3, 10), (1, 3)]

QA_ROWS_PER_STEP = 6


def _table_body(e0, e1, e2, e3, e4, e5, e6, e7, w_ref, b_ref, out_ref,
                qa_ref, qb_ref):
    es = [e0, e1, e2, e3, e4, e5, e6, e7]

    @pl.when(pl.program_id(0) == 0)
    def _build_quads():
        ps = []
        for t in range(8):
            ps.append(jnp.dot(es[t][...], w_ref[WOFFS[t]:WOFFS[t] + WIDTHS[t], :],
                              preferred_element_type=jnp.float32))
        qa = jnp.broadcast_to(b_ref[...], (NA, 128))
        for t, (stride, voc) in enumerate(ADIGS):
            r = lax.broadcasted_iota(jnp.int32, (NA, VOCABS[t]), 0)
            c = lax.broadcasted_iota(jnp.int32, (NA, VOCABS[t]), 1)
            sel = ((r // stride) % voc == c).astype(jnp.float32)
            qa = qa + jnp.dot(sel, ps[t], preferred_element_type=jnp.float32)
        qa_ref[...] = qa
        qb = jnp.zeros((NB_, 128), jnp.float32)
        for k, (stride, voc) in enumerate(BDIGS):
            t = 4 + k
            r = lax.broadcasted_iota(jnp.int32, (NB_, VOCABS[t]), 0)
            c = lax.broadcasted_iota(jnp.int32, (NB_, VOCABS[t]), 1)
            sel = ((r // stride) % voc == c).astype(jnp.float32)
            qb = qb + jnp.dot(sel, ps[t], preferred_element_type=jnp.float32)
        qb_ref[...] = qb

    i = pl.program_id(0)
    qa_rows = qa_ref[pl.ds(i * QA_ROWS_PER_STEP, QA_ROWS_PER_STEP), :]
    out_ref[...] = qa_rows[:, None, :] + qb_ref[...][None, :, :]


def _build_table(es, W, b2):
    in_specs = []
    for t in range(8):
        in_specs.append(pl.BlockSpec((VOCABS[t], WIDTHS[t]), lambda i: (0, 0)))
    in_specs.append(pl.BlockSpec((104, 128), lambda i: (0, 0)))
    in_specs.append(pl.BlockSpec((1, 128), lambda i: (0, 0)))
    return pl.pallas_call(
        _table_body,
        grid=(NA // QA_ROWS_PER_STEP,),
        in_specs=in_specs,
        out_specs=pl.BlockSpec((QA_ROWS_PER_STEP, NB_, 128), lambda i: (i, 0, 0)),
        out_shape=jax.ShapeDtypeStruct((NA, NB_, 128), jnp.float32),
        scratch_shapes=[pltpu.VMEM((NA, 128), jnp.float32),
                        pltpu.VMEM((NB_, 128), jnp.float32)],
        compiler_params=pltpu.CompilerParams(
            dimension_semantics=("arbitrary",)),
    )(*es, W, b2)


IDX_TN = 16384


def _idx_body(tok_ref, out_ref):
    his = [s & ~0x3F for s in STRIDES]
    los = [s & 0x3F for s in STRIDES]
    tokf = tok_ref[...].astype(jnp.float32)
    r8 = lax.broadcasted_iota(jnp.int32, (8, 1), 0)
    sva = jnp.full((8, 1), float(his[7]), jnp.float32)
    svb = jnp.full((8, 1), float(los[7]), jnp.float32)
    for t in range(7):
        sva = jnp.where(r8 == t, float(his[t]), sva)
        svb = jnp.where(r8 == t, float(los[t]), svb)
    idxf = (jnp.dot(tokf, sva, preferred_element_type=jnp.float32)
            + jnp.dot(tokf, svb, preferred_element_type=jnp.float32))
    out_ref[...] = (idxf + 0.5).astype(jnp.int32)


def _build_idx(tok2):
    return pl.pallas_call(
        _idx_body,
        grid=(BS // IDX_TN,),
        in_specs=[pl.BlockSpec((IDX_TN, 8), lambda i: (i, 0))],
        out_specs=pl.BlockSpec((IDX_TN, 1), lambda i: (i, 0)),
        out_shape=jax.ShapeDtypeStruct((BS, 1), jnp.int32),
    )(tok2)


NC, NS = 2, 16
NW = NC * NS
CH = 128
CHUNKS = 4
OUTER = CH * CHUNKS
TOK_PER_W = BS // NW
NOUT = TOK_PER_W // OUTER


def _sc_body(bt_hbm, idx_hbm, out_hbm,
             idx_a, idx_b, rows_v,
             isem_a, isem_b, sg0, sg1, sg2, sg3, so0, so1, so2, so3):
    sg = [sg0, sg1, sg2, sg3]
    so = [so0, so1, so2, so3]
    idxs = [idx_a, idx_b]
    isem = [isem_a, isem_b]
    wid = lax.axis_index("s") * NC + lax.axis_index("c")
    tok0 = wid * TOK_PER_W

    def idx_copy(row_t, p):
        return pltpu.make_async_copy(
            idx_hbm.at[pl.ds(row_t, CHUNKS), :], idxs[p], isem[p])

    def gather_copy(p, j):
        return pltpu.make_async_copy(
            bt_hbm.at[idxs[p].at[j]], rows_v.at[pl.ds(j * CH, CH), :], sg[j])

    def out_copy(base_t, j):
        return pltpu.make_async_copy(
            rows_v.at[pl.ds(j * CH, CH), :],
            out_hbm.at[pl.ds(base_t + j * CH, CH), :], so[j])

    row0 = wid * (TOK_PER_W // CH)
    idx_copy(row0, 0).start()

    def outer_body(g2, carry):
        for p in range(2):
            go = g2 * 2 + p
            base_t = tok0 + go * OUTER
            idx_copy(row0 + go * CHUNKS, p).wait()

            @pl.when(go + 1 < NOUT)
            def _prefetch_idx():
                idx_copy(row0 + (go + 1) * CHUNKS, 1 - p).start()

            for j in range(CHUNKS):
                @pl.when(go > 0)
                def _slot_free(j=j):
                    out_copy(base_t - OUTER, j).wait()
                gather_copy(p, j).start()
            for j in range(CHUNKS):
                gather_copy(p, j).wait()
                out_copy(base_t, j).start()
        return carry

    lax.fori_loop(0, NOUT // 2, outer_body, 0)
    last_t = tok0 + (NOUT - 1) * OUTER
    for j in range(CHUNKS):
        out_copy(last_t, j).wait()


def _sc_gather(bt, idx1):
    mesh = plsc.VectorSubcoreMesh(core_axis_name="c", subcore_axis_name="s",
                                  num_cores=NC, num_subcores=NS)
    return pl.kernel(
        _sc_body,
        out_type=jax.ShapeDtypeStruct((BS, 128), jnp.float32),
        mesh=mesh,
        scratch_types=[pltpu.VMEM((CHUNKS, CH), jnp.int32),
                       pltpu.VMEM((CHUNKS, CH), jnp.int32),
                       pltpu.VMEM((CHUNKS * CH, 128), jnp.float32),
                       pltpu.SemaphoreType.DMA,
                       pltpu.SemaphoreType.DMA,
                       pltpu.SemaphoreType.DMA,
                       pltpu.SemaphoreType.DMA,
                       pltpu.SemaphoreType.DMA,
                       pltpu.SemaphoreType.DMA,
                       pltpu.SemaphoreType.DMA,
                       pltpu.SemaphoreType.DMA,
                       pltpu.SemaphoreType.DMA,
                       pltpu.SemaphoreType.DMA],
    )(bt, idx1)


def kernel(tokens, E0, E1, E2, E3, E4, E5, E6, E7, W, b):
    es = [E0, E1, E2, E3, E4, E5, E6, E7]
    bt = _build_table(es, W, b.reshape(1, 128)).reshape(NA * NB_, 128)
    idx = _build_idx(tokens.reshape(BS, 8)).reshape(BS // CH, CH)
    out = _sc_gather(bt, idx)
    return out.reshape(B, S, 128)


# speedup vs baseline: 1.1717x; 1.0071x over previous
import jax
import jax.numpy as jnp
from jax import lax
from jax.experimental import pallas as pl
from jax.experimental.pallas import tpu as pltpu
from jax.experimental.pallas import tpu_sc as plsc

B, S = 4096, 200
BS = B * S

VOCABS = [7, 7, 2, 3, 4, 2, 10, 3]
WIDTHS = [16, 16, 8, 8, 16, 8, 16, 16]
WOFFS = [0, 16, 32, 40, 48, 64, 72, 88]
NA, NB_ = 294, 240
STRIDES = [42 * NB_, 6 * NB_, 3 * NB_, NB_, 60, 30, 3, 1]
ADIGS = [(42, 7), (6, 7), (3, 2), (1, 3)]
BDIGS = [(60, 4), (30, 2), (3, 10), (1, 3)]

QA_ROWS_PER_STEP = 6


def _table_body(e0, e1, e2, e3, e4, e5, e6, e7, w_ref, b_ref, out_ref,
                qa_ref, qb_ref):
    es = [e0, e1, e2, e3, e4, e5, e6, e7]

    @pl.when(pl.program_id(0) == 0)
    def _build_quads():
        ps = []
        for t in range(8):
            ps.append(jnp.dot(es[t][...], w_ref[WOFFS[t]:WOFFS[t] + WIDTHS[t], :],
                              preferred_element_type=jnp.float32))
        qa = jnp.broadcast_to(b_ref[...], (NA, 128))
        for t, (stride, voc) in enumerate(ADIGS):
            r = lax.broadcasted_iota(jnp.int32, (NA, VOCABS[t]), 0)
            c = lax.broadcasted_iota(jnp.int32, (NA, VOCABS[t]), 1)
            sel = ((r // stride) % voc == c).astype(jnp.float32)
            qa = qa + jnp.dot(sel, ps[t], preferred_element_type=jnp.float32)
        qa_ref[...] = qa
        qb = jnp.zeros((NB_, 128), jnp.float32)
        for k, (stride, voc) in enumerate(BDIGS):
            t = 4 + k
            r = lax.broadcasted_iota(jnp.int32, (NB_, VOCABS[t]), 0)
            c = lax.broadcasted_iota(jnp.int32, (NB_, VOCABS[t]), 1)
            sel = ((r // stride) % voc == c).astype(jnp.float32)
            qb = qb + jnp.dot(sel, ps[t], preferred_element_type=jnp.float32)
        qb_ref[...] = qb

    i = pl.program_id(0)
    qa_rows = qa_ref[pl.ds(i * QA_ROWS_PER_STEP, QA_ROWS_PER_STEP), :]
    out_ref[...] = qa_rows[:, None, :] + qb_ref[...][None, :, :]


def _build_table(es, W, b2):
    in_specs = []
    for t in range(8):
        in_specs.append(pl.BlockSpec((VOCABS[t], WIDTHS[t]), lambda i: (0, 0)))
    in_specs.append(pl.BlockSpec((104, 128), lambda i: (0, 0)))
    in_specs.append(pl.BlockSpec((1, 128), lambda i: (0, 0)))
    return pl.pallas_call(
        _table_body,
        grid=(NA // QA_ROWS_PER_STEP,),
        in_specs=in_specs,
        out_specs=pl.BlockSpec((QA_ROWS_PER_STEP, NB_, 128), lambda i: (i, 0, 0)),
        out_shape=jax.ShapeDtypeStruct((NA, NB_, 128), jnp.float32),
        scratch_shapes=[pltpu.VMEM((NA, 128), jnp.float32),
                        pltpu.VMEM((NB_, 128), jnp.float32)],
        compiler_params=pltpu.CompilerParams(
            dimension_semantics=("arbitrary",)),
    )(*es, W, b2)


IDX_TR = 2048


def _idx_body(tok_ref, out_ref):
    tokf = tok_ref[...].astype(jnp.float32)
    l = lax.broadcasted_iota(jnp.int32, (128, 16), 0)
    g = lax.broadcasted_iota(jnp.int32, (128, 16), 1)
    seg = (l // 8 == g)
    sl = l % 8
    idxf = jnp.zeros((IDX_TR, 16), jnp.float32)
    for mask in (~0x3F, 0x3F):
        sv = jnp.zeros((128, 16), jnp.float32)
        for t in range(8):
            part = float(STRIDES[t] & mask)
            sv = jnp.where(seg & (sl == t), part, sv)
        idxf = idxf + jnp.dot(tokf, sv, preferred_element_type=jnp.float32)
    out_ref[...] = (idxf + 0.5).astype(jnp.int32)


def _build_idx(tok16):
    return pl.pallas_call(
        _idx_body,
        grid=(BS // 16 // IDX_TR,),
        in_specs=[pl.BlockSpec((IDX_TR, 128), lambda i: (i, 0))],
        out_specs=pl.BlockSpec((IDX_TR, 16), lambda i: (i, 0)),
        out_shape=jax.ShapeDtypeStruct((BS // 16, 16), jnp.int32),
    )(tok16)


NC, NS = 2, 16
NW = NC * NS
CH = 128
CHUNKS = 5
OUTER = CH * CHUNKS
TOK_PER_W = BS // NW
NOUT = TOK_PER_W // OUTER


IDXG = 8
IDX_ROWS = IDXG * CHUNKS
NGRP = NOUT // IDXG


def _sc_body(bt_hbm, idx_hbm, out_hbm, idx_v, rows_v,
             sg0, sg1, sg2, sg3, sg4, so0, so1, so2, so3, so4):
    sg = [sg0, sg1, sg2, sg3, sg4]
    so = [so0, so1, so2, so3, so4]
    wid = lax.axis_index("s") * NC + lax.axis_index("c")
    tok0 = wid * TOK_PER_W
    row0 = wid * (TOK_PER_W // CH)

    def gather_copy(q, j):
        return pltpu.make_async_copy(
            bt_hbm.at[idx_v.at[q * CHUNKS + j]],
            rows_v.at[pl.ds(j * CH, CH), :], sg[j])

    def out_copy(base_t, j):
        return pltpu.make_async_copy(
            rows_v.at[pl.ds(j * CH, CH), :],
            out_hbm.at[pl.ds(base_t + j * CH, CH), :], so[j])

    def grp_body(g, carry):
        pltpu.sync_copy(idx_hbm.at[pl.ds(row0 + g * IDX_ROWS, IDX_ROWS), :],
                        idx_v)
        for q in range(IDXG):
            go = g * IDXG + q
            base_t = tok0 + go * OUTER
            for j in range(CHUNKS):
                if q == 0:
                    @pl.when(g > 0)
                    def _slot_free(j=j, base_t=base_t):
                        out_copy(base_t - OUTER, j).wait()
                else:
                    out_copy(base_t - OUTER, j).wait()
                gather_copy(q, j).start()
            for j in range(CHUNKS):
                gather_copy(q, j).wait()
                out_copy(base_t, j).start()
        return carry

    lax.fori_loop(0, NGRP, grp_body, 0)
    last_t = tok0 + (NOUT - 1) * OUTER
    for j in range(CHUNKS):
        out_copy(last_t, j).wait()


def _sc_gather(bt, idx1):
    mesh = plsc.VectorSubcoreMesh(core_axis_name="c", subcore_axis_name="s",
                                  num_cores=NC, num_subcores=NS)
    return pl.kernel(
        _sc_body,
        out_type=jax.ShapeDtypeStruct((BS, 128), jnp.float32),
        mesh=mesh,
        scratch_types=[pltpu.VMEM((IDX_ROWS, CH), jnp.int32),
                       pltpu.VMEM((CHUNKS * CH, 128), jnp.float32),
                       pltpu.SemaphoreType.DMA,
                       pltpu.SemaphoreType.DMA,
                       pltpu.SemaphoreType.DMA,
                       pltpu.SemaphoreType.DMA,
                       pltpu.SemaphoreType.DMA,
                       pltpu.SemaphoreType.DMA,
                       pltpu.SemaphoreType.DMA,
                       pltpu.SemaphoreType.DMA,
                       pltpu.SemaphoreType.DMA,
                       pltpu.SemaphoreType.DMA],
    )(bt, idx1)


def kernel(tokens, E0, E1, E2, E3, E4, E5, E6, E7, W, b):
    es = [E0, E1, E2, E3, E4, E5, E6, E7]
    bt = _build_table(es, W, b.reshape(1, 128)).reshape(NA * NB_, 128)
    idx = _build_idx(tokens.reshape(BS // 16, 128)).reshape(BS // CH, CH)
    out = _sc_gather(bt, idx)
    return out.reshape(B, S, 128)
